# baseline (device time: 34894 ns/iter reference)
import jax
import jax.numpy as jnp
from jax import lax
from jax.experimental import pallas as pl
from jax.experimental.pallas import tpu as pltpu


def kernel(partial, resid, gamma):
    _, m, d = partial.shape
    gamma2d = gamma.reshape(1, d)

    def body(p_ref, r_ref, g_ref, o_ref, send_buf, recv_buf, send_sem, recv_sem):
        my_x = lax.axis_index("x")
        my_y = lax.axis_index("y")
        my_z = lax.axis_index("z")
        nbr = (1 - my_x, my_y, my_z)

        send_buf[...] = p_ref[0].astype(jnp.bfloat16)

        barrier_sem = pltpu.get_barrier_semaphore()
        pl.semaphore_signal(
            barrier_sem, inc=1, device_id=nbr,
            device_id_type=pl.DeviceIdType.MESH,
        )
        pl.semaphore_wait(barrier_sem, 1)

        rdma = pltpu.make_async_remote_copy(
            src_ref=send_buf,
            dst_ref=recv_buf,
            send_sem=send_sem,
            recv_sem=recv_sem,
            device_id=nbr,
            device_id_type=pl.DeviceIdType.MESH,
        )
        rdma.start()
        rdma.wait()

        y = p_ref[0] + recv_buf[...].astype(jnp.float32) + r_ref[...]
        rms = jnp.sqrt(jnp.mean(y * y, axis=-1, keepdims=True) + 1e-6)
        o_ref[...] = (y / rms) * g_ref[...]

    return pl.pallas_call(
        body,
        out_shape=jax.ShapeDtypeStruct((m, d), jnp.float32),
        in_specs=[pl.BlockSpec(memory_space=pltpu.VMEM)] * 3,
        out_specs=pl.BlockSpec(memory_space=pltpu.VMEM),
        scratch_shapes=[
            pltpu.VMEM((m, d), jnp.bfloat16),
            pltpu.VMEM((m, d), jnp.bfloat16),
            pltpu.SemaphoreType.DMA,
            pltpu.SemaphoreType.DMA,
        ],
        compiler_params=pltpu.CompilerParams(collective_id=0),
    )(partial, resid, gamma2d)


# device time: 26656 ns/iter; 1.3090x vs baseline; 1.3090x over previous
import jax
import jax.numpy as jnp
from jax import lax
from jax.experimental import pallas as pl
from jax.experimental.pallas import tpu as pltpu

C = 8


def kernel(partial, resid, gamma):
    _, m, d = partial.shape
    half = m // 2
    ch = half // C
    gamma2d = gamma.reshape(1, d)

    def body(p_ref, r_ref, g_ref, o_ref, stage, other,
             xsend_sems, xrecv_sems, zsend_sems, zrecv_sems):
        my_x = lax.axis_index("x")
        my_y = lax.axis_index("y")
        my_z = lax.axis_index("z")
        xnbr = (1 - my_x, my_y, my_z)
        znbr = (my_x, my_y, 1 - my_z)
        half_off = my_z * half

        for c in range(C):
            stage[c] = p_ref[0, pl.ds(half_off + c * ch, ch), :].astype(
                jnp.bfloat16
            )

        barrier_sem = pltpu.get_barrier_semaphore()
        for nbr in (xnbr, znbr):
            pl.semaphore_signal(
                barrier_sem, inc=1, device_id=nbr,
                device_id_type=pl.DeviceIdType.MESH,
            )
        pl.semaphore_wait(barrier_sem, 2)

        xr = []
        for c in range(C):
            rdma = pltpu.make_async_remote_copy(
                src_ref=stage.at[c],
                dst_ref=other.at[my_z * C + c],
                send_sem=xsend_sems.at[c],
                recv_sem=xrecv_sems.at[c],
                device_id=xnbr,
                device_id_type=pl.DeviceIdType.MESH,
            )
            rdma.start()
            xr.append(rdma)

        def ln_chunk(j, row0):
            y = (
                p_ref[0, pl.ds(row0, ch), :]
                + other[j].astype(jnp.float32)
                + r_ref[pl.ds(row0, ch), :]
            )
            rms = jnp.sqrt(jnp.mean(y * y, axis=-1, keepdims=True) + 1e-6)
            o_ref[pl.ds(row0, ch), :] = (y / rms) * g_ref[...]

        fwd = []
        for c in range(C):
            xr[c].wait_recv()
            j = my_z * C + c
            f = pltpu.make_async_remote_copy(
                src_ref=other.at[j],
                dst_ref=other.at[j],
                send_sem=zsend_sems.at[c],
                recv_sem=zrecv_sems.at[c],
                device_id=znbr,
                device_id_type=pl.DeviceIdType.MESH,
            )
            f.start()
            fwd.append(f)
            ln_chunk(j, half_off + c * ch)

        for c in range(C):
            jr = (1 - my_z) * C + c
            recv = pltpu.make_async_remote_copy(
                src_ref=other.at[jr],
                dst_ref=other.at[jr],
                send_sem=zsend_sems.at[c],
                recv_sem=zrecv_sems.at[c],
                device_id=znbr,
                device_id_type=pl.DeviceIdType.MESH,
            )
            recv.wait_recv()
            ln_chunk(jr, (1 - my_z) * half + c * ch)

        for c in range(C):
            xr[c].wait_send()
            fwd[c].wait_send()

    return pl.pallas_call(
        body,
        out_shape=jax.ShapeDtypeStruct((m, d), jnp.float32),
        in_specs=[pl.BlockSpec(memory_space=pltpu.VMEM)] * 3,
        out_specs=pl.BlockSpec(memory_space=pltpu.VMEM),
        scratch_shapes=[
            pltpu.VMEM((C, ch, d), jnp.bfloat16),
            pltpu.VMEM((2 * C, ch, d), jnp.bfloat16),
            pltpu.SemaphoreType.DMA((C,)),
            pltpu.SemaphoreType.DMA((C,)),
            pltpu.SemaphoreType.DMA((C,)),
            pltpu.SemaphoreType.DMA((C,)),
        ],
        compiler_params=pltpu.CompilerParams(collective_id=0),
    )(partial, resid, gamma2d)


# device time: 26579 ns/iter; 1.3128x vs baseline; 1.0029x over previous
import jax
import jax.numpy as jnp
from jax import lax
from jax.experimental import pallas as pl
from jax.experimental.pallas import tpu as pltpu

C = 8


def kernel(partial, resid, gamma):
    _, m, d = partial.shape
    half = m // 2
    ch = half // C
    gamma2d = gamma.reshape(1, d)

    def body(p_ref, r_ref, g_ref, o_ref, stage, other, outstage, zin,
             xsend_sems, xrecv_sems, zsend_sems, zrecv_sems):
        my_x = lax.axis_index("x")
        my_y = lax.axis_index("y")
        my_z = lax.axis_index("z")
        xnbr = (1 - my_x, my_y, my_z)
        znbr = (my_x, my_y, 1 - my_z)
        half_off = my_z * half

        for c in range(C):
            stage[c] = p_ref[0, pl.ds(half_off + c * ch, ch), :].astype(
                jnp.bfloat16
            )

        barrier_sem = pltpu.get_barrier_semaphore()
        for nbr in (xnbr, znbr):
            pl.semaphore_signal(
                barrier_sem, inc=1, device_id=nbr,
                device_id_type=pl.DeviceIdType.MESH,
            )
        pl.semaphore_wait(barrier_sem, 2)

        xr = []
        for c in range(C):
            rdma = pltpu.make_async_remote_copy(
                src_ref=stage.at[c],
                dst_ref=other.at[c],
                send_sem=xsend_sems.at[c],
                recv_sem=xrecv_sems.at[c],
                device_id=xnbr,
                device_id_type=pl.DeviceIdType.MESH,
            )
            rdma.start()
            xr.append(rdma)

        fwd = []
        for c in range(C):
            xr[c].wait_recv()
            row0 = half_off + c * ch
            y = (
                p_ref[0, pl.ds(row0, ch), :]
                + other[c].astype(jnp.float32)
                + r_ref[pl.ds(row0, ch), :]
            )
            rms = jnp.sqrt(jnp.mean(y * y, axis=-1, keepdims=True) + 1e-6)
            out_bf = ((y / rms) * g_ref[...]).astype(jnp.bfloat16)
            outstage[c] = out_bf
            o_ref[pl.ds(row0, ch), :] = out_bf.astype(jnp.float32)
            f = pltpu.make_async_remote_copy(
                src_ref=outstage.at[c],
                dst_ref=zin.at[c],
                send_sem=zsend_sems.at[c],
                recv_sem=zrecv_sems.at[c],
                device_id=znbr,
                device_id_type=pl.DeviceIdType.MESH,
            )
            f.start()
            fwd.append(f)

        for c in range(C):
            recv = pltpu.make_async_remote_copy(
                src_ref=zin.at[c],
                dst_ref=zin.at[c],
                send_sem=zsend_sems.at[c],
                recv_sem=zrecv_sems.at[c],
                device_id=znbr,
                device_id_type=pl.DeviceIdType.MESH,
            )
            recv.wait_recv()
            row0 = (1 - my_z) * half + c * ch
            o_ref[pl.ds(row0, ch), :] = zin[c].astype(jnp.float32)

        for c in range(C):
            xr[c].wait_send()
            fwd[c].wait_send()

    return pl.pallas_call(
        body,
        out_shape=jax.ShapeDtypeStruct((m, d), jnp.float32),
        in_specs=[pl.BlockSpec(memory_space=pltpu.VMEM)] * 3,
        out_specs=pl.BlockSpec(memory_space=pltpu.VMEM),
        scratch_shapes=[
            pltpu.VMEM((C, ch, d), jnp.bfloat16),
            pltpu.VMEM((C, ch, d), jnp.bfloat16),
            pltpu.VMEM((C, ch, d), jnp.bfloat16),
            pltpu.VMEM((C, ch, d), jnp.bfloat16),
            pltpu.SemaphoreType.DMA((C,)),
            pltpu.SemaphoreType.DMA((C,)),
            pltpu.SemaphoreType.DMA((C,)),
            pltpu.SemaphoreType.DMA((C,)),
        ],
        compiler_params=pltpu.CompilerParams(collective_id=0),
    )(partial, resid, gamma2d)


# device time: 23394 ns/iter; 1.4916x vs baseline; 1.1361x over previous
import jax
import jax.numpy as jnp
from jax import lax
from jax.experimental import pallas as pl
from jax.experimental.pallas import tpu as pltpu

K = 8


def kernel(partial, resid, gamma):
    _, m, d = partial.shape
    q = m // 4
    ch = q // K
    H = K // 2
    gamma2d = gamma.reshape(1, d)

    def body(p_ref, r_ref, g_ref, o_ref,
             stage, other, outstage, yin, zin, diagin, loc,
             xs_s, xs_r, ys_s, ys_r, zs_s, zs_r, dy_s, dy_r, dz_s, dz_r):
        my_x = lax.axis_index("x")
        my_y = lax.axis_index("y")
        my_z = lax.axis_index("z")
        xnbr = (1 - my_x, my_y, my_z)
        ynbr = (my_x, 1 - my_y, my_z)
        znbr = (my_x, my_y, 1 - my_z)

        q_own = 2 * my_y + my_z
        q_y = 2 * (1 - my_y) + my_z
        q_z = 2 * my_y + (1 - my_z)
        q_d = 2 * (1 - my_y) + (1 - my_z)
        row_own = q_own * q

        barrier_sem = pltpu.get_barrier_semaphore()
        pl.semaphore_signal(
            barrier_sem, inc=4, device_id=xnbr,
            device_id_type=pl.DeviceIdType.MESH,
        )
        for nbr in (ynbr, znbr):
            pl.semaphore_signal(
                barrier_sem, inc=1, device_id=nbr,
                device_id_type=pl.DeviceIdType.MESH,
            )

        for k in range(K):
            stage[k] = p_ref[0, pl.ds(row_own + k * ch, ch), :].astype(
                jnp.bfloat16
            )

        loc[...] = (
            p_ref[0, pl.ds(row_own, q), :] + r_ref[pl.ds(row_own, q), :]
        )

        pl.semaphore_wait(barrier_sem, 4)

        def copy(src, dst, ssem, rsem, dev):
            return pltpu.make_async_remote_copy(
                src_ref=src, dst_ref=dst, send_sem=ssem, recv_sem=rsem,
                device_id=dev, device_id_type=pl.DeviceIdType.MESH,
            )

        xr = [copy(stage.at[k], other.at[k], xs_s.at[k], xs_r.at[k], xnbr)
              for k in range(K)]
        for r in xr:
            r.start()

        ysend, zsend = [], []
        for k in range(K):
            xr[k].wait_recv()
            if k == 0:
                pl.semaphore_wait(barrier_sem, 2)
            row0 = row_own + k * ch
            y = loc[pl.ds(k * ch, ch), :] + other[k].astype(jnp.float32)
            rms = jnp.sqrt(jnp.mean(y * y, axis=-1, keepdims=True) + 1e-6)
            out_bf = ((y / rms) * g_ref[...]).astype(jnp.bfloat16)
            outstage[k] = out_bf
            o_ref[pl.ds(row0, ch), :] = out_bf.astype(jnp.float32)
            fy = copy(outstage.at[k], yin.at[k], ys_s.at[k], ys_r.at[k], ynbr)
            fz = copy(outstage.at[k], zin.at[k], zs_s.at[k], zs_r.at[k], znbr)
            fy.start()
            fz.start()
            ysend.append(fy)
            zsend.append(fz)

        dfwd = []
        for k in range(H):
            zsend[k].wait_recv()
            f = copy(zin.at[k], diagin.at[k], dy_s.at[k], dy_r.at[k], ynbr)
            f.start()
            dfwd.append(f)
            o_ref[pl.ds(q_z * q + k * ch, ch), :] = zin[k].astype(jnp.float32)
        for k in range(H, K):
            ysend[k].wait_recv()
            f = copy(yin.at[k], diagin.at[k], dz_s.at[k - H], dz_r.at[k - H],
                     znbr)
            f.start()
            dfwd.append(f)
            o_ref[pl.ds(q_y * q + k * ch, ch), :] = yin[k].astype(jnp.float32)

        for k in range(H):
            ysend[k].wait_recv()
            o_ref[pl.ds(q_y * q + k * ch, ch), :] = yin[k].astype(jnp.float32)
        for k in range(H, K):
            zsend[k].wait_recv()
            o_ref[pl.ds(q_z * q + k * ch, ch), :] = zin[k].astype(jnp.float32)

        for k in range(K):
            if k < H:
                recv = copy(diagin.at[k], diagin.at[k], dy_s.at[k],
                            dy_r.at[k], ynbr)
            else:
                recv = copy(diagin.at[k], diagin.at[k], dz_s.at[k - H],
                            dz_r.at[k - H], znbr)
            recv.wait_recv()
            o_ref[pl.ds(q_d * q + k * ch, ch), :] = diagin[k].astype(
                jnp.float32
            )

        for k in range(K):
            xr[k].wait_send()
            ysend[k].wait_send()
            zsend[k].wait_send()
        for f in dfwd:
            f.wait_send()

    return pl.pallas_call(
        body,
        out_shape=jax.ShapeDtypeStruct((m, d), jnp.float32),
        in_specs=[pl.BlockSpec(memory_space=pltpu.VMEM)] * 3,
        out_specs=pl.BlockSpec(memory_space=pltpu.VMEM),
        scratch_shapes=[
            pltpu.VMEM((K, ch, d), jnp.bfloat16),
            pltpu.VMEM((K, ch, d), jnp.bfloat16),
            pltpu.VMEM((K, ch, d), jnp.bfloat16),
            pltpu.VMEM((K, ch, d), jnp.bfloat16),
            pltpu.VMEM((K, ch, d), jnp.bfloat16),
            pltpu.VMEM((K, ch, d), jnp.bfloat16),
            pltpu.VMEM((q, d), jnp.float32),
            pltpu.SemaphoreType.DMA((K,)),
            pltpu.SemaphoreType.DMA((K,)),
            pltpu.SemaphoreType.DMA((K,)),
            pltpu.SemaphoreType.DMA((K,)),
            pltpu.SemaphoreType.DMA((K,)),
            pltpu.SemaphoreType.DMA((K,)),
            pltpu.SemaphoreType.DMA((H,)),
            pltpu.SemaphoreType.DMA((H,)),
            pltpu.SemaphoreType.DMA((H,)),
            pltpu.SemaphoreType.DMA((H,)),
        ],
        compiler_params=pltpu.CompilerParams(collective_id=0),
    )(partial, resid, gamma2d)
